# Initial kernel scaffold; baseline (speedup 1.0000x reference)
import jax
import jax.numpy as jnp
from jax.experimental import pallas as pl


def kernel(x, edge_index, batch, W1_0, b1_0, W2_0, b2_0, g_0, be_0, W1_1, b1_1, W2_1, b2_1, g_1, be_1, Wc, bc):
    raise NotImplementedError("write your pallas kernel here")


# speedup vs baseline: 5.2677x; 5.2677x over previous
import functools

import jax
import jax.numpy as jnp
from jax import lax
from jax.experimental import pallas as pl
from jax.experimental.pallas import tpu as pltpu
from jax.experimental.pallas import tpu_sc as plsc

N = 10000
D = 128
H = 128
C = 40
G = 128
E = 320000

NC = 2
NS = 16
HF = H // NC
CH = 128
NCH = -(-E // (NS * CH))
EPW = NCH * CH
EPAD = NS * EPW
ACC_ROWS = 10240
RPT = ACC_ROWS // NS
ZR = 128


def _edge_agg_body(x_hbm, src_hbm, dst_hbm, out_hbm,
                   src_v, dst_v, rowbuf, zbuf, acc_sh, sem0):
    c = lax.axis_index("c")
    s = lax.axis_index("s")

    def _zrow(r, _):
        for k in range(HF // 16):
            zbuf[r, pl.ds(k * 16, 16)] = jnp.zeros((16,), jnp.float32)
        return 0
    lax.fori_loop(0, ZR, _zrow, 0)
    for r in range(RPT // ZR):
        pltpu.sync_copy(zbuf, acc_sh.at[pl.ds(s * RPT + r * ZR, ZR)])

    pltpu.sync_copy(src_hbm.at[s], src_v)
    pltpu.sync_copy(dst_hbm.at[s], dst_v)

    plsc.subcore_barrier()

    def _chunk(j, _):
        pltpu.async_copy(x_hbm.at[c].at[src_v.at[j]], rowbuf, sem0).wait()
        pltpu.sync_copy(rowbuf, acc_sh.at[dst_v.at[j]], add=True)
        return 0
    lax.fori_loop(0, NCH, _chunk, 0)

    plsc.subcore_barrier()

    row0 = s * RPT

    @pl.when(s < NS - 1)
    def _():
        pltpu.sync_copy(acc_sh.at[pl.ds(row0, RPT)],
                        out_hbm.at[c, pl.ds(row0, RPT)])

    @pl.when(s == NS - 1)
    def _():
        pltpu.sync_copy(acc_sh.at[pl.ds((NS - 1) * RPT, N - (NS - 1) * RPT)],
                        out_hbm.at[c, pl.ds((NS - 1) * RPT, N - (NS - 1) * RPT)])


_edge_agg = functools.partial(
    pl.kernel,
    out_type=jax.ShapeDtypeStruct((NC, N, HF), jnp.float32),
    mesh=plsc.VectorSubcoreMesh(core_axis_name="c", subcore_axis_name="s"),
    scratch_types=[
        pltpu.VMEM((NCH, CH), jnp.int32),
        pltpu.VMEM((NCH, CH), jnp.int32),
        pltpu.VMEM((CH, HF), jnp.float32),
        pltpu.VMEM((ZR, HF), jnp.float32),
        pltpu.VMEM_SHARED((ACC_ROWS, HF), jnp.float32),
        pltpu.SemaphoreType.DMA,
    ],
    compiler_params=pltpu.CompilerParams(use_tc_tiling_on_sc=False),
)(_edge_agg_body)


def _mlp_bn(xs, a, w1, b1, w2, b2, g, be):
    z = (jnp.concatenate([xs[0], xs[1]], axis=-1)
         + jnp.concatenate([a[0], a[1]], axis=-1))
    z = jnp.maximum(
        jnp.dot(z, w1, preferred_element_type=jnp.float32) + b1, 0.0)
    z = jnp.dot(z, w2, preferred_element_type=jnp.float32) + b2
    m = jnp.mean(z, axis=0, keepdims=True)
    d = z - m
    v = jnp.mean(d * d, axis=0, keepdims=True)
    return d * lax.rsqrt(v + 1e-5) * g + be


def _layer_body(x_ref, a_ref, w1_ref, b1_ref, w2_ref, b2_ref, g_ref, be_ref,
                o_ref):
    h = _mlp_bn(x_ref[...], a_ref[...], w1_ref[...], b1_ref[...], w2_ref[...],
                b2_ref[...], g_ref[...], be_ref[...])
    h = jnp.maximum(h, 0.0)
    o_ref[...] = jnp.stack([h[:, :HF], h[:, HF:]], axis=0)


def _final_body(x_ref, a_ref, w1_ref, b1_ref, w2_ref, b2_ref, g_ref, be_ref,
                batch_ref, wc_ref, bc_ref, o_ref):
    h = _mlp_bn(x_ref[...], a_ref[...], w1_ref[...], b1_ref[...], w2_ref[...],
                b2_ref[...], g_ref[...], be_ref[...])
    oh = (lax.broadcasted_iota(jnp.int32, (G, 1), 0)
          == batch_ref[...]).astype(jnp.float32)
    pooled = jnp.dot(oh, h, preferred_element_type=jnp.float32)
    o_ref[...] = (jnp.dot(pooled, wc_ref[...], preferred_element_type=jnp.float32)
                  + bc_ref[...])


_TC_PARAMS = pltpu.CompilerParams(vmem_limit_bytes=100 * 1024 * 1024)


def _layer_call(xs, agg, w1, b1, w2, b2, g, be):
    return pl.pallas_call(
        _layer_body,
        out_shape=jax.ShapeDtypeStruct((NC, N, HF), jnp.float32),
        compiler_params=_TC_PARAMS,
    )(xs, agg, w1, b1.reshape(1, H), w2, b2.reshape(1, H),
      g.reshape(1, H), be.reshape(1, H))


def _final_call(xs, agg, w1, b1, w2, b2, g, be, batch, wc, bc):
    return pl.pallas_call(
        _final_body,
        out_shape=jax.ShapeDtypeStruct((G, C), jnp.float32),
        compiler_params=_TC_PARAMS,
    )(xs, agg, w1, b1.reshape(1, H), w2, b2.reshape(1, H),
      g.reshape(1, H), be.reshape(1, H), batch.reshape(1, N), wc,
      bc.reshape(1, C))


def kernel(x, edge_index, batch, W1_0, b1_0, W2_0, b2_0, g_0, be_0,
           W1_1, b1_1, W2_1, b2_1, g_1, be_1, Wc, bc):
    src = edge_index[0]
    dst = edge_index[1]
    srcp = jnp.concatenate(
        [src, jnp.zeros((EPAD - E,), jnp.int32)]).reshape(NS, NCH, CH)
    dstp = jnp.concatenate(
        [dst, jnp.full((EPAD - E,), N, jnp.int32)]).reshape(NS, NCH, CH)
    xs = jnp.stack([x[:, :HF], x[:, HF:]], axis=0)

    agg0 = _edge_agg(xs, srcp, dstp)
    h1s = _layer_call(xs, agg0, W1_0, b1_0, W2_0, b2_0, g_0, be_0)
    agg1 = _edge_agg(h1s, srcp, dstp)
    return _final_call(h1s, agg1, W1_1, b1_1, W2_1, b2_1, g_1, be_1,
                       batch, Wc, bc)


# speedup vs baseline: 7.6837x; 1.4587x over previous
import functools

import jax
import jax.numpy as jnp
from jax import lax
from jax.experimental import pallas as pl
from jax.experimental.pallas import tpu as pltpu
from jax.experimental.pallas import tpu_sc as plsc

N = 10000
D = 128
H = 128
C = 40
G = 128
E = 320000

NC = 2
NS = 16
HF = H // NC
CH = 128
NCH = -(-E // (NS * CH))
EPW = NCH * CH
EPAD = NS * EPW
ACC_ROWS = 10240
RPT = ACC_ROWS // NS
ZR = 128


def _edge_agg_body(x_hbm, src_hbm, dst_hbm, out_hbm,
                   src_v, dst_v, rowa, rowb, zbuf, acc_sh, sema, semb):
    c = lax.axis_index("c")
    s = lax.axis_index("s")

    def _zrow(r, _):
        for k in range(HF // 16):
            zbuf[r, pl.ds(k * 16, 16)] = jnp.zeros((16,), jnp.float32)
        return 0
    lax.fori_loop(0, ZR, _zrow, 0)
    for r in range(RPT // ZR):
        pltpu.sync_copy(zbuf, acc_sh.at[pl.ds(s * RPT + r * ZR, ZR)])

    pltpu.sync_copy(src_hbm.at[s], src_v)
    pltpu.sync_copy(dst_hbm.at[s], dst_v)

    plsc.subcore_barrier()

    def _gather(j, buf, sem):
        pltpu.async_copy(x_hbm.at[c].at[src_v.at[j]], buf, sem)

    def _wait(buf, sem):
        pltpu.make_async_copy(x_hbm.at[c].at[src_v.at[0]], buf, sem).wait()

    def _scat(j, buf):
        pltpu.sync_copy(buf, acc_sh.at[dst_v.at[j]], add=True)

    _gather(0, rowa, sema)

    def _pair(j2, _):
        a = 2 * j2
        _gather(a + 1, rowb, semb)
        _wait(rowa, sema)
        _scat(a, rowa)
        _gather(a + 2, rowa, sema)
        _wait(rowb, semb)
        _scat(a + 1, rowb)
        return 0
    lax.fori_loop(0, (NCH - 1) // 2, _pair, 0)
    _wait(rowa, sema)
    _scat(NCH - 1, rowa)

    plsc.subcore_barrier()

    row0 = s * RPT

    @pl.when(s < NS - 1)
    def _():
        pltpu.sync_copy(acc_sh.at[pl.ds(row0, RPT)],
                        out_hbm.at[c, pl.ds(row0, RPT)])

    @pl.when(s == NS - 1)
    def _():
        pltpu.sync_copy(acc_sh.at[pl.ds((NS - 1) * RPT, N - (NS - 1) * RPT)],
                        out_hbm.at[c, pl.ds((NS - 1) * RPT, N - (NS - 1) * RPT)])


_edge_agg = functools.partial(
    pl.kernel,
    out_type=jax.ShapeDtypeStruct((NC, N, HF), jnp.float32),
    mesh=plsc.VectorSubcoreMesh(core_axis_name="c", subcore_axis_name="s"),
    scratch_types=[
        pltpu.VMEM((NCH, CH), jnp.int32),
        pltpu.VMEM((NCH, CH), jnp.int32),
        pltpu.VMEM((CH, HF), jnp.float32),
        pltpu.VMEM((CH, HF), jnp.float32),
        pltpu.VMEM((ZR, HF), jnp.float32),
        pltpu.VMEM_SHARED((ACC_ROWS, HF), jnp.float32),
        pltpu.SemaphoreType.DMA,
        pltpu.SemaphoreType.DMA,
    ],
    compiler_params=pltpu.CompilerParams(use_tc_tiling_on_sc=False),
)(_edge_agg_body)


def _mlp_bn(xs, a, w1, b1, w2, b2, g, be):
    z = (jnp.concatenate([xs[0], xs[1]], axis=-1)
         + jnp.concatenate([a[0], a[1]], axis=-1))
    z = jnp.maximum(
        jnp.dot(z, w1, preferred_element_type=jnp.float32) + b1, 0.0)
    z = jnp.dot(z, w2, preferred_element_type=jnp.float32) + b2
    m = jnp.mean(z, axis=0, keepdims=True)
    d = z - m
    v = jnp.mean(d * d, axis=0, keepdims=True)
    return d * lax.rsqrt(v + 1e-5) * g + be


def _layer_body(x_ref, a_ref, w1_ref, b1_ref, w2_ref, b2_ref, g_ref, be_ref,
                o_ref):
    h = _mlp_bn(x_ref[...], a_ref[...], w1_ref[...], b1_ref[...], w2_ref[...],
                b2_ref[...], g_ref[...], be_ref[...])
    h = jnp.maximum(h, 0.0)
    o_ref[...] = jnp.stack([h[:, :HF], h[:, HF:]], axis=0)


def _final_body(x_ref, a_ref, w1_ref, b1_ref, w2_ref, b2_ref, g_ref, be_ref,
                batch_ref, wc_ref, bc_ref, o_ref):
    h = _mlp_bn(x_ref[...], a_ref[...], w1_ref[...], b1_ref[...], w2_ref[...],
                b2_ref[...], g_ref[...], be_ref[...])
    oh = (lax.broadcasted_iota(jnp.int32, (G, 1), 0)
          == batch_ref[...]).astype(jnp.float32)
    pooled = jnp.dot(oh, h, preferred_element_type=jnp.float32)
    o_ref[...] = (jnp.dot(pooled, wc_ref[...], preferred_element_type=jnp.float32)
                  + bc_ref[...])


_TC_PARAMS = pltpu.CompilerParams(vmem_limit_bytes=100 * 1024 * 1024)


def _layer_call(xs, agg, w1, b1, w2, b2, g, be):
    return pl.pallas_call(
        _layer_body,
        out_shape=jax.ShapeDtypeStruct((NC, N, HF), jnp.float32),
        compiler_params=_TC_PARAMS,
    )(xs, agg, w1, b1.reshape(1, H), w2, b2.reshape(1, H),
      g.reshape(1, H), be.reshape(1, H))


def _final_call(xs, agg, w1, b1, w2, b2, g, be, batch, wc, bc):
    return pl.pallas_call(
        _final_body,
        out_shape=jax.ShapeDtypeStruct((G, C), jnp.float32),
        compiler_params=_TC_PARAMS,
    )(xs, agg, w1, b1.reshape(1, H), w2, b2.reshape(1, H),
      g.reshape(1, H), be.reshape(1, H), batch.reshape(1, N), wc,
      bc.reshape(1, C))


def kernel(x, edge_index, batch, W1_0, b1_0, W2_0, b2_0, g_0, be_0,
           W1_1, b1_1, W2_1, b2_1, g_1, be_1, Wc, bc):
    src = edge_index[0]
    dst = edge_index[1]
    srcp = jnp.concatenate(
        [src, jnp.zeros((EPAD - E,), jnp.int32)]).reshape(NS, NCH, CH)
    dstp = jnp.concatenate(
        [dst, jnp.full((EPAD - E,), N, jnp.int32)]).reshape(NS, NCH, CH)
    xs = jnp.stack([x[:, :HF], x[:, HF:]], axis=0)

    agg0 = _edge_agg(xs, srcp, dstp)
    h1s = _layer_call(xs, agg0, W1_0, b1_0, W2_0, b2_0, g_0, be_0)
    agg1 = _edge_agg(h1s, srcp, dstp)
    return _final_call(h1s, agg1, W1_1, b1_1, W2_1, b2_1, g_1, be_1,
                       batch, Wc, bc)
